# baseline (speedup 1.0000x reference)
import functools

import jax
import jax.numpy as jnp
from jax import lax
from jax.experimental import pallas as pl
from jax.experimental.pallas import tpu as pltpu
from jax.experimental.pallas import tpu_sc as plsc

_BATCH = 16384
_D = 64
_NC = 2
_NS = 16
_NW = _NC * _NS
_BPW = _BATCH // _NW
_CH = 128
_NCH = _BPW // _CH


def _gather_body(idx_hbm, ut_hbm, it_hbm, u_hbm, p_hbm, n_hbm, idx_v, rows_v, sem):
    wid = lax.axis_index("s") * _NC + lax.axis_index("c")
    base = wid * _BPW
    pltpu.sync_copy(idx_hbm.at[wid], idx_v)

    tables = (ut_hbm, it_hbm, it_hbm)

    copies = []
    for c in range(3):
        for j in range(_NCH):
            copies.append(
                pltpu.async_copy(
                    tables[c].at[idx_v.at[c, j]],
                    rows_v.at[c, pl.ds(j * _CH, _CH)],
                    sem,
                )
            )
    for cp in copies:
        cp.wait()

    outs = (u_hbm, p_hbm, n_hbm)
    for c in range(3):
        pltpu.sync_copy(rows_v.at[c], outs[c].at[pl.ds(base, _BPW)])


_mesh = plsc.VectorSubcoreMesh(core_axis_name="c", subcore_axis_name="s")

_gather_call = functools.partial(
    pl.kernel,
    out_type=[jax.ShapeDtypeStruct((_BATCH, _D), jnp.float32)] * 3,
    mesh=_mesh,
    scratch_types=[
        pltpu.VMEM((3, _NCH, _CH), jnp.int32),
        pltpu.VMEM((3, _BPW, _D), jnp.float32),
        pltpu.SemaphoreType.DMA,
    ],
    compiler_params=pltpu.CompilerParams(use_tc_tiling_on_sc=False),
)(_gather_body)


def kernel(user, pos, neg, user_table, item_table):
    idx = jnp.stack(
        [
            user.reshape(_NW, _NCH, _CH),
            pos.reshape(_NW, _NCH, _CH),
            neg.reshape(_NW, _NCH, _CH),
        ],
        axis=1,
    )
    u_e, p_e, n_e = _gather_call(idx, user_table, item_table)
    return jnp.concatenate([u_e, p_e, n_e], axis=-1).reshape(
        _BATCH, 1, 3 * _D
    )


# speedup vs baseline: 1.0010x; 1.0010x over previous
import functools

import jax
import jax.numpy as jnp
from jax import lax
from jax.experimental import pallas as pl
from jax.experimental.pallas import tpu as pltpu
from jax.experimental.pallas import tpu_sc as plsc

_BATCH = 16384
_D = 64
_NC = 2
_NS = 16
_NW = _NC * _NS
_BPW = _BATCH // _NW
_CH = 128
_NCH = _BPW // _CH


def _gather_body(idx_hbm, ut_hbm, it_hbm, u_hbm, p_hbm, n_hbm,
                 idx_v, stage_v, sem):
    wid = lax.axis_index("s") * _NC + lax.axis_index("c")
    base = wid * _BPW
    pltpu.sync_copy(idx_hbm.at[wid], idx_v)

    tables = (ut_hbm, it_hbm, it_hbm)
    outs = (u_hbm, p_hbm, n_hbm)

    copies = []
    for c in range(3):
        for j in range(_NCH):
            copies.append(
                pltpu.async_copy(
                    tables[c].at[idx_v.at[c, j]],
                    stage_v.at[c, pl.ds(j * _CH, _CH)],
                    sem,
                )
            )
    for cp in copies:
        cp.wait()

    for c in range(3):
        pltpu.sync_copy(stage_v.at[c], outs[c].at[pl.ds(base, _BPW)])


_mesh = plsc.VectorSubcoreMesh(core_axis_name="c", subcore_axis_name="s")

_gather_call = functools.partial(
    pl.kernel,
    out_type=[jax.ShapeDtypeStruct((_BATCH, _D), jnp.float32)] * 3,
    mesh=_mesh,
    scratch_types=[
        pltpu.VMEM((3, _NCH, _CH), jnp.int32),
        pltpu.VMEM((3, _BPW, _D), jnp.float32),
        pltpu.SemaphoreType.DMA,
    ],
    compiler_params=pltpu.CompilerParams(use_tc_tiling_on_sc=False),
)(_gather_body)


def kernel(user, pos, neg, user_table, item_table):
    idx = jnp.stack(
        [
            user.reshape(_NW, _NCH, _CH),
            pos.reshape(_NW, _NCH, _CH),
            neg.reshape(_NW, _NCH, _CH),
        ],
        axis=1,
    )
    u_e, p_e, n_e = _gather_call(idx, user_table, item_table)
    return jnp.concatenate([u_e, p_e, n_e], axis=-1).reshape(
        _BATCH, 1, 3 * _D
    )
